# Initial kernel scaffold; baseline (speedup 1.0000x reference)
import jax
import jax.numpy as jnp
from jax.experimental import pallas as pl


def kernel(x, M, w_I, b_I, gamma, beta):
    raise NotImplementedError("write your pallas kernel here")


# speedup vs baseline: 1.2544x; 1.2544x over previous
import functools

import jax
import jax.numpy as jnp
import numpy as np
from jax import lax
from jax.experimental import pallas as pl
from jax.experimental.pallas import tpu as pltpu


def _conv_stats_kernel(m_ref, x_ref, w1_ref, w2_ref, w3_ref, b_ref, cmask_ref,
                       z_ref, mout_ref, s1_ref, s2_ref,
                       *, Wp, L2, LP, LQ):
    f32 = jnp.float32
    mslab = m_ref[0]
    msum = jnp.zeros((1, LP), f32)
    for kh in range(3):
        for kw in range(3):
            s = kh * Wp + kw
            msum = msum + mslab[:, s:s + LP]
    m1 = jnp.where(msum == 0.0, 1.0, msum)

    xs = x_ref[0]
    P = (m1 * xs.astype(f32)).astype(jnp.bfloat16)

    Q1 = jnp.concatenate([P[:, :LQ], P[:, 1:LQ + 1]], axis=0)
    QW = jnp.concatenate([P[:, :LQ], P[:, Wp:Wp + LQ]], axis=0)
    acc = jnp.dot(w1_ref[0], Q1[:, 0:L2], preferred_element_type=f32)
    acc = acc + jnp.dot(w1_ref[1], Q1[:, Wp:Wp + L2], preferred_element_type=f32)
    acc = acc + jnp.dot(w1_ref[2], Q1[:, 2 * Wp:2 * Wp + L2], preferred_element_type=f32)
    acc = acc + jnp.dot(w2_ref[...], QW[:, 2:2 + L2], preferred_element_type=f32)
    acc = acc + jnp.dot(w3_ref[...], P[:, 2 * Wp + 2:2 * Wp + 2 + L2],
                        preferred_element_type=f32)
    y = acc + b_ref[...]

    off = Wp + 1
    inv_m = 1.0 / m1[:, off:off + L2]
    z = y * inv_m

    z_ref[0] = z.astype(jnp.bfloat16)
    mout_ref[0] = msum[:, off:off + L2]

    zm = z * cmask_ref[...]
    s1_ref[0] = jnp.sum(zm, axis=1, keepdims=True)
    s2_ref[0] = jnp.sum(zm * z, axis=1, keepdims=True)


def _bn_relu_kernel(z_ref, mo_ref, a_ref, b_ref, out_ref, mout_ref, *, W):
    zv = z_ref[0][:, :, :W].astype(jnp.float32)
    out_ref[0] = jnp.maximum(zv * a_ref[...] + b_ref[...], 0.0)
    mout_ref[0] = mo_ref[0][:, :, :W]


def kernel(x, M, w_I, b_I, gamma, beta):
    N, Cin, H, W = x.shape
    Cout = w_I.shape[0]
    eps = 1e-5
    f32 = jnp.float32
    bf16 = jnp.bfloat16

    Wp = W + 2
    L2 = H * Wp
    LP = (H + 5) * Wp
    LQ = (H + 3) * Wp
    Lm = 2 * Wp + 3 + LP

    xb = x.astype(f32).astype(bf16)
    x_flat = jnp.pad(xb, ((0, 0), (0, 0), (1, 4), (1, 1))).reshape(N, Cin, LP)

    Mf = M.astype(f32)
    m_flat = jnp.pad(Mf, ((0, 0), (0, 0), (2, 2), (1, 1))).reshape(N, 1, (H + 4) * Wp)
    m_flat = jnp.pad(m_flat, ((0, 0), (0, 0), (1, Lm - 1 - (H + 4) * Wp)))

    w_tap = w_I.astype(f32).transpose(2, 3, 0, 1).reshape(9, Cout, Cin)
    w1 = jnp.concatenate([w_tap[0::3], w_tap[1::3]], axis=2).astype(bf16)
    w2 = jnp.concatenate([w_tap[2], w_tap[5]], axis=1).astype(bf16)
    w3 = w_tap[8].astype(bf16)
    bias = b_I.astype(f32).reshape(Cout, 1)

    idx = np.arange(L2)
    cmask = jnp.asarray((idx % Wp < W).astype(np.float32)).reshape(1, L2)

    cparams = pltpu.CompilerParams(
        dimension_semantics=("parallel",),
        vmem_limit_bytes=64 * 1024 * 1024,
    )

    kern1 = functools.partial(_conv_stats_kernel, Wp=Wp, L2=L2, LP=LP, LQ=LQ)
    z, mo_flat, ssum, ssq = pl.pallas_call(
        kern1,
        grid=(N,),
        in_specs=[
            pl.BlockSpec((1, 1, Lm), lambda g: (g, 0, 0)),
            pl.BlockSpec((1, Cin, LP), lambda g: (g, 0, 0)),
            pl.BlockSpec((3, Cout, 2 * Cin), lambda g: (0, 0, 0)),
            pl.BlockSpec((Cout, 2 * Cin), lambda g: (0, 0)),
            pl.BlockSpec((Cout, Cin), lambda g: (0, 0)),
            pl.BlockSpec((Cout, 1), lambda g: (0, 0)),
            pl.BlockSpec((1, L2), lambda g: (0, 0)),
        ],
        out_specs=(
            pl.BlockSpec((1, Cout, L2), lambda g: (g, 0, 0)),
            pl.BlockSpec((1, 1, L2), lambda g: (g, 0, 0)),
            pl.BlockSpec((1, Cout, 1), lambda g: (g, 0, 0)),
            pl.BlockSpec((1, Cout, 1), lambda g: (g, 0, 0)),
        ),
        out_shape=(
            jax.ShapeDtypeStruct((N, Cout, L2), bf16),
            jax.ShapeDtypeStruct((N, 1, L2), f32),
            jax.ShapeDtypeStruct((N, Cout, 1), f32),
            jax.ShapeDtypeStruct((N, Cout, 1), f32),
        ),
        compiler_params=cparams,
    )(m_flat, x_flat, w1, w2, w3, bias, cmask)

    cnt = float(N * H * W)
    mean = jnp.sum(ssum, axis=0)[:, 0] / cnt
    var = jnp.maximum(jnp.sum(ssq, axis=0)[:, 0] / cnt - mean * mean, 0.0)
    a = gamma.astype(f32) * lax.rsqrt(var + eps)
    bshift = beta.astype(f32) - a * mean
    a = a.reshape(Cout, 1, 1)
    bshift = bshift.reshape(Cout, 1, 1)

    z4 = z.reshape(N, Cout, H, Wp)
    mo4 = mo_flat.reshape(N, 1, H, Wp)

    kern2 = functools.partial(_bn_relu_kernel, W=W)
    x_out, m_out = pl.pallas_call(
        kern2,
        grid=(N,),
        in_specs=[
            pl.BlockSpec((1, Cout, H, Wp), lambda g: (g, 0, 0, 0)),
            pl.BlockSpec((1, 1, H, Wp), lambda g: (g, 0, 0, 0)),
            pl.BlockSpec((Cout, 1, 1), lambda g: (0, 0, 0)),
            pl.BlockSpec((Cout, 1, 1), lambda g: (0, 0, 0)),
        ],
        out_specs=(
            pl.BlockSpec((1, Cout, H, W), lambda g: (g, 0, 0, 0)),
            pl.BlockSpec((1, 1, H, W), lambda g: (g, 0, 0, 0)),
        ),
        out_shape=(
            jax.ShapeDtypeStruct((N, Cout, H, W), f32),
            jax.ShapeDtypeStruct((N, 1, H, W), f32),
        ),
        compiler_params=cparams,
    )(z4, mo4, a, bshift)

    return x_out, m_out
